# baseline (speedup 1.0000x reference)
import functools

import jax
import jax.numpy as jnp
from jax import lax
from jax.experimental import pallas as pl
from jax.experimental.pallas import tpu as pltpu

N = 1024
H = 4
R = 1024
KB = N // R
C1, C2 = 16, 8
F1, F2 = H * C1, H * C2


def _prologue_compute(feats, A, C, fout,
                      w_ref, v_ref, q_ref, fbd_ref, cm_ref):
    sall = jnp.dot(feats, A, preferred_element_type=jnp.float32)
    sallT = jnp.transpose(sall)
    slT = sallT[:H, :]
    srT = sallT[H:, :]
    msr = jnp.max(srT, axis=1, keepdims=True)
    xm = slT + msr
    wT = jnp.exp(jnp.minimum(-0.8 * xm, 40.0))
    w_ref[...] = jnp.transpose(wT)
    srm = srT - msr
    v_ref[...] = jnp.exp(srm)
    q_ref[...] = jnp.exp(0.2 * srm)
    cm_ref[...] = jnp.dot(jnp.full((1, N), 1.0 / N, dtype=jnp.float32),
                          feats, preferred_element_type=jnp.float32)
    blocks = []
    for h in range(H):
        hm = ((lax.broadcasted_iota(jnp.int32, (1, fout), 1) // C) == h)
        fb_h = feats * hm.astype(jnp.float32)
        oh = (lax.broadcasted_iota(jnp.int32, (1, H), 1) == h)
        ones_h = jnp.broadcast_to(oh.astype(jnp.float32), (N, H))
        blocks.append(jnp.concatenate([fb_h, ones_h], axis=1))
    fbd_ref[...] = jnp.concatenate(blocks, axis=0)


def _attn_rows(i0, maskf, C, fout, w_ref, v_ref, q_ref, fbd_ref, cm_ref):
    O = jnp.zeros((R, fout + H), dtype=jnp.float32)
    for h in range(H):
        wh = w_ref[pl.ds(i0, R), h:h + 1]
        vh = v_ref[h:h + 1, :]
        qh = q_ref[h:h + 1, :]
        eh = jnp.maximum(vh, wh * qh) * maskf
        O = O + jnp.dot(eh, fbd_ref[pl.ds(h * N, N), :],
                        preferred_element_type=jnp.float32)
    outs = []
    for h in range(H):
        s = O[:, fout + h:fout + h + 1]
        num = O[:, h * C:(h + 1) * C]
        cm = cm_ref[0:1, h * C:(h + 1) * C]
        s_safe = jnp.where(s > 0, s, 1.0)
        outs.append(jnp.where(s > 0, num / s_safe, cm))
    return jnp.concatenate(outs, axis=1)


def _fused_body(x_ref, adj_ref, W1_ref, b1_ref, A1_ref,
                W2_ref, b2_ref, A2_ref, out_ref,
                w1_ref, v1_ref, q1_ref, fbd1_ref, cm1_ref,
                w2_ref, v2_ref, q2_ref, fbd2_ref, cm2_ref,
                mask_ref, h1_ref):
    i = pl.program_id(0)

    @pl.when(i == 0)
    def _prologue1():
        x = x_ref[...]
        feats = lax.dot_general(x, W1_ref[...], (((1,), (1,)), ((), ())),
                                preferred_element_type=jnp.float32)
        feats = feats + b1_ref[...][None, :]
        _prologue_compute(feats, A1_ref[...], C1, F1,
                          w1_ref, v1_ref, q1_ref, fbd1_ref, cm1_ref)

    @pl.when(i == KB)
    def _prologue2():
        h1 = h1_ref[...]
        feats = lax.dot_general(h1, W2_ref[...], (((1,), (1,)), ((), ())),
                                preferred_element_type=jnp.float32)
        feats = feats + b2_ref[...][None, :]
        _prologue_compute(feats, A2_ref[...], C2, F2,
                          w2_ref, v2_ref, q2_ref, fbd2_ref, cm2_ref)

    @pl.when(i < KB)
    def _layer1_step():
        adjb = adj_ref[...]
        maskf = (adjb != 0).astype(jnp.float32)
        mask_ref[pl.ds(i * R, R), :] = maskf
        res = _attn_rows(i * R, maskf, C1, F1,
                         w1_ref, v1_ref, q1_ref, fbd1_ref, cm1_ref)
        res = jnp.where(res > 0, res, jnp.exp(res) - 1.0)
        h1_ref[pl.ds(i * R, R), :] = res

    @pl.when(i >= KB)
    def _layer2_step():
        i0 = (i - KB) * R
        maskf = mask_ref[pl.ds(i0, R), :]
        res = _attn_rows(i0, maskf, C2, F2,
                         w2_ref, v2_ref, q2_ref, fbd2_ref, cm2_ref)
        mm = jnp.max(res, axis=1, keepdims=True)
        ee = jnp.exp(res - mm)
        out_ref[pl.ds(i0, R), :] = ee / jnp.sum(ee, axis=1, keepdims=True)


def _head_proj(a, C):
    fout = H * C
    eye = jnp.eye(H, dtype=a.dtype)
    Al = (a[:, :C, None] * eye[:, None, :]).reshape(fout, H)
    Ar = (a[:, C:, None] * eye[:, None, :]).reshape(fout, H)
    return jnp.concatenate([Al, Ar], axis=1)


@jax.jit
def kernel(obs, adj_matrix, W1, b1, a1, W2, b2, a2):
    x = obs.reshape(N, -1)
    adj = adj_matrix.reshape(N, N)
    fin = x.shape[1]
    A1 = _head_proj(a1, C1)
    A2 = _head_proj(a2, C2)
    return pl.pallas_call(
        _fused_body,
        grid=(2 * KB,),
        in_specs=[
            pl.BlockSpec((N, fin), lambda i: (0, 0)),
            pl.BlockSpec((R, N), lambda i: (jnp.where(i < KB, i, 0), 0)),
            pl.BlockSpec((F1, fin), lambda i: (0, 0)),
            pl.BlockSpec((F1,), lambda i: (0,)),
            pl.BlockSpec((F1, 2 * H), lambda i: (0, 0)),
            pl.BlockSpec((F2, F1), lambda i: (0, 0)),
            pl.BlockSpec((F2,), lambda i: (0,)),
            pl.BlockSpec((F2, 2 * H), lambda i: (0, 0)),
        ],
        out_specs=pl.BlockSpec((N, F2), lambda i: (0, 0)),
        out_shape=jax.ShapeDtypeStruct((N, F2), jnp.float32),
        scratch_shapes=[
            pltpu.VMEM((N, H), jnp.float32),
            pltpu.VMEM((H, N), jnp.float32),
            pltpu.VMEM((H, N), jnp.float32),
            pltpu.VMEM((H * N, F1 + H), jnp.float32),
            pltpu.VMEM((1, F1), jnp.float32),
            pltpu.VMEM((N, H), jnp.float32),
            pltpu.VMEM((H, N), jnp.float32),
            pltpu.VMEM((H, N), jnp.float32),
            pltpu.VMEM((H * N, F2 + H), jnp.float32),
            pltpu.VMEM((1, F2), jnp.float32),
            pltpu.VMEM((N, N), jnp.float32),
            pltpu.VMEM((N, F1), jnp.float32),
        ],
        compiler_params=pltpu.CompilerParams(
            dimension_semantics=("arbitrary",),
        ),
    )(x, adj, W1, b1, A1, W2, b2, A2)


# speedup vs baseline: 1.2020x; 1.2020x over previous
import functools

import jax
import jax.numpy as jnp
from jax import lax
from jax.experimental import pallas as pl
from jax.experimental.pallas import tpu as pltpu

N = 1024
H = 4
R = 512
KB = N // R
C1, C2 = 16, 8
F1, F2 = H * C1, H * C2


def _prologue_compute(feats, A, C, fout,
                      w_ref, v_ref, q_ref, fbd_ref, cm_ref):
    sall = jnp.dot(feats, A, preferred_element_type=jnp.float32)
    sallT = jnp.transpose(sall)
    slT = sallT[:H, :]
    srT = sallT[H:, :]
    msr = jnp.max(srT, axis=1, keepdims=True)
    xm = slT + msr
    wT = jnp.exp(jnp.minimum(-0.8 * xm, 40.0))
    w_ref[...] = jnp.transpose(wT)
    srm = srT - msr
    v_ref[...] = jnp.exp(srm)
    q_ref[...] = jnp.exp(0.2 * srm)
    cm_ref[...] = jnp.dot(jnp.full((1, N), 1.0 / N, dtype=jnp.float32),
                          feats, preferred_element_type=jnp.float32)
    blocks = []
    for h in range(H):
        hm = ((lax.broadcasted_iota(jnp.int32, (1, fout), 1) // C) == h)
        fb_h = feats * hm.astype(jnp.float32)
        oh = (lax.broadcasted_iota(jnp.int32, (1, H), 1) == h)
        ones_h = jnp.broadcast_to(oh.astype(jnp.float32), (N, H))
        blocks.append(jnp.concatenate([fb_h, ones_h], axis=1))
    fbd_ref[...] = jnp.concatenate(blocks, axis=0)


def _attn_rows(i0, maskf, C, fout, w_ref, v_ref, q_ref, fbd_ref, cm_ref):
    O = jnp.zeros((R, fout + H), dtype=jnp.float32)
    for h in range(H):
        wh = w_ref[pl.ds(i0, R), h:h + 1]
        vh = v_ref[h:h + 1, :]
        qh = q_ref[h:h + 1, :]
        eh = jnp.maximum(vh, wh * qh) * maskf
        O = O + jnp.dot(eh, fbd_ref[pl.ds(h * N, N), :],
                        preferred_element_type=jnp.float32)
    rowi = lax.broadcasted_iota(jnp.int32, (fout + H, fout), 0)
    coli = lax.broadcasted_iota(jnp.int32, (fout + H, fout), 1)
    P = ((rowi - fout) == (coli // C)).astype(jnp.float32)
    sfull = jnp.dot(O, P, preferred_element_type=jnp.float32)
    pos = sfull > 0
    return jnp.where(pos, O[:, :fout] / jnp.where(pos, sfull, 1.0),
                     cm_ref[...])


def _fused_body(x_ref, adj_ref, W1_ref, b1_ref, A1_ref,
                W2_ref, b2_ref, A2_ref, out_ref,
                w1_ref, v1_ref, q1_ref, fbd1_ref, cm1_ref,
                w2_ref, v2_ref, q2_ref, fbd2_ref, cm2_ref,
                mask_ref, h1_ref):
    i = pl.program_id(0)

    @pl.when(i == 0)
    def _prologue1():
        x = x_ref[...]
        feats = lax.dot_general(x, W1_ref[...], (((1,), (1,)), ((), ())),
                                preferred_element_type=jnp.float32)
        feats = feats + b1_ref[...][None, :]
        _prologue_compute(feats, A1_ref[...], C1, F1,
                          w1_ref, v1_ref, q1_ref, fbd1_ref, cm1_ref)

    @pl.when(i == KB)
    def _prologue2():
        h1 = h1_ref[...]
        feats = lax.dot_general(h1, W2_ref[...], (((1,), (1,)), ((), ())),
                                preferred_element_type=jnp.float32)
        feats = feats + b2_ref[...][None, :]
        _prologue_compute(feats, A2_ref[...], C2, F2,
                          w2_ref, v2_ref, q2_ref, fbd2_ref, cm2_ref)

    @pl.when(i < KB)
    def _layer1_step():
        adjb = adj_ref[...]
        maskf = (adjb != 0).astype(jnp.float32)
        mask_ref[pl.ds(i * R, R), :] = maskf
        res = _attn_rows(i * R, maskf, C1, F1,
                         w1_ref, v1_ref, q1_ref, fbd1_ref, cm1_ref)
        res = jnp.where(res > 0, res, jnp.exp(res) - 1.0)
        h1_ref[pl.ds(i * R, R), :] = res

    @pl.when(i >= KB)
    def _layer2_step():
        i0 = (i - KB) * R
        maskf = mask_ref[pl.ds(i0, R), :]
        res = _attn_rows(i0, maskf, C2, F2,
                         w2_ref, v2_ref, q2_ref, fbd2_ref, cm2_ref)
        mm = jnp.max(res, axis=1, keepdims=True)
        ee = jnp.exp(res - mm)
        out_ref[pl.ds(i0, R), :] = ee / jnp.sum(ee, axis=1, keepdims=True)


def _head_proj(a, C):
    fout = H * C
    eye = jnp.eye(H, dtype=a.dtype)
    Al = (a[:, :C, None] * eye[:, None, :]).reshape(fout, H)
    Ar = (a[:, C:, None] * eye[:, None, :]).reshape(fout, H)
    return jnp.concatenate([Al, Ar], axis=1)


@jax.jit
def kernel(obs, adj_matrix, W1, b1, a1, W2, b2, a2):
    x = obs.reshape(N, -1)
    adj = adj_matrix.reshape(N, N)
    fin = x.shape[1]
    A1 = _head_proj(a1, C1)
    A2 = _head_proj(a2, C2)
    return pl.pallas_call(
        _fused_body,
        grid=(2 * KB,),
        in_specs=[
            pl.BlockSpec((N, fin), lambda i: (0, 0)),
            pl.BlockSpec((R, N), lambda i: (jnp.where(i < KB, i, 0), 0)),
            pl.BlockSpec((F1, fin), lambda i: (0, 0)),
            pl.BlockSpec((F1,), lambda i: (0,)),
            pl.BlockSpec((F1, 2 * H), lambda i: (0, 0)),
            pl.BlockSpec((F2, F1), lambda i: (0, 0)),
            pl.BlockSpec((F2,), lambda i: (0,)),
            pl.BlockSpec((F2, 2 * H), lambda i: (0, 0)),
        ],
        out_specs=pl.BlockSpec((N, F2), lambda i: (0, 0)),
        out_shape=jax.ShapeDtypeStruct((N, F2), jnp.float32),
        scratch_shapes=[
            pltpu.VMEM((N, H), jnp.float32),
            pltpu.VMEM((H, N), jnp.float32),
            pltpu.VMEM((H, N), jnp.float32),
            pltpu.VMEM((H * N, F1 + H), jnp.float32),
            pltpu.VMEM((1, F1), jnp.float32),
            pltpu.VMEM((N, H), jnp.float32),
            pltpu.VMEM((H, N), jnp.float32),
            pltpu.VMEM((H, N), jnp.float32),
            pltpu.VMEM((H * N, F2 + H), jnp.float32),
            pltpu.VMEM((1, F2), jnp.float32),
            pltpu.VMEM((N, N), jnp.float32),
            pltpu.VMEM((N, F1), jnp.float32),
        ],
        compiler_params=pltpu.CompilerParams(
            dimension_semantics=("arbitrary",),
        ),
    )(x, adj, W1, b1, A1, W2, b2, A2)
